# baseline (speedup 1.0000x reference)
import jax
import jax.numpy as jnp
from jax import lax
from jax.experimental import pallas as pl
from jax.experimental.pallas import tpu as pltpu

MEMORY_LEN = 100000
FEAT = 128
T = 50

_ROWS = MEMORY_LEN // FEAT
_MAIN = _ROWS * FEAT
_TAIL0 = MEMORY_LEN - FEAT
_BIG = 2**30


def _queue_kernel(x_ref, idx_ref, tail_ref, mem_ref, mem_out_ref,
                  idx_out_ref, scratch_ref, sem):
    xs = x_ref[:]
    reward_sum = jnp.sum(xs[:, FEAT - 1:FEAT])
    states = xs[T - 1:T, :]

    mainv = idx_ref[:]
    tailv = tail_ref[:]
    pos_m = (lax.broadcasted_iota(jnp.int32, mainv.shape, 0) * FEAT
             + lax.broadcasted_iota(jnp.int32, mainv.shape, 1))
    pos_t = _TAIL0 + lax.broadcasted_iota(jnp.int32, tailv.shape, 1)

    min_val = jnp.minimum(jnp.min(mainv), jnp.min(tailv))
    max_m = jnp.max(mainv)
    max_t = jnp.max(tailv)
    max_val = jnp.maximum(max_m, max_t)
    mp_m = jnp.min(jnp.where(mainv == max_val, pos_m, _BIG))
    mp_t = jnp.min(jnp.where(tailv == max_val, pos_t, _BIG))
    max_pos = jnp.minimum(mp_m, mp_t)
    am = jnp.where(max_val > min_val, max_pos, 1)

    cp = pltpu.make_async_copy(
        mem_ref.at[pl.ds(am, 1), :], scratch_ref, sem)
    cp.start()
    cp.wait()

    use_new = reward_sum > max_val
    mem_out_ref[:] = jnp.where(use_new, states, scratch_ref[:])
    idx_out_ref[:] = jnp.full((1, 1), jnp.where(use_new, reward_sum, max_val),
                              dtype=jnp.float32)


@jax.jit
def kernel(x, maximum_route, memory, index):
    del maximum_route
    xs = x.reshape(T, FEAT)
    idx = index.reshape(MEMORY_LEN)
    idx_main = idx[:_MAIN].reshape(_ROWS, FEAT)
    idx_tail = idx[_TAIL0:].reshape(1, FEAT)
    mem = memory.reshape(MEMORY_LEN, FEAT)

    mem_out, idx_out = pl.pallas_call(
        _queue_kernel,
        in_specs=[
            pl.BlockSpec(memory_space=pltpu.VMEM),
            pl.BlockSpec(memory_space=pltpu.VMEM),
            pl.BlockSpec(memory_space=pltpu.VMEM),
            pl.BlockSpec(memory_space=pl.ANY),
        ],
        out_specs=[
            pl.BlockSpec(memory_space=pltpu.VMEM),
            pl.BlockSpec(memory_space=pltpu.VMEM),
        ],
        out_shape=[
            jax.ShapeDtypeStruct((1, FEAT), jnp.float32),
            jax.ShapeDtypeStruct((1, 1), jnp.float32),
        ],
        scratch_shapes=[
            pltpu.VMEM((1, FEAT), jnp.float32),
            pltpu.SemaphoreType.DMA,
        ],
    )(xs, idx_main, idx_tail, mem)

    return mem_out.reshape(1, 1, FEAT), idx_out.reshape(1, 1, 1)


# speedup vs baseline: 1.3444x; 1.3444x over previous
import jax
import jax.numpy as jnp
from jax import lax
from jax.experimental import pallas as pl
from jax.experimental.pallas import tpu as pltpu

MEMORY_LEN = 100000
FEAT = 128
T = 50

_ROWS = (MEMORY_LEN + FEAT - 1) // FEAT
_PAD = _ROWS * FEAT - MEMORY_LEN
_BIG = 2**30


def _queue_kernel(x_ref, idx_ref, mem_ref, mem_out_ref, idx_out_ref,
                  scratch_ref, sem):
    xs = x_ref[:]
    reward_sum = jnp.sum(xs[:, FEAT - 1:FEAT])
    states = xs[T - 1:T, :]

    idxv = idx_ref[:]
    pos = (lax.broadcasted_iota(jnp.int32, idxv.shape, 0) * FEAT
           + lax.broadcasted_iota(jnp.int32, idxv.shape, 1))
    valid = pos < MEMORY_LEN

    min_val = jnp.min(idxv)
    vmax = jnp.where(valid, idxv, -jnp.inf)
    max_val = jnp.max(vmax)
    max_pos = jnp.min(jnp.where(vmax == max_val, pos, _BIG))
    am = jnp.where(max_val > min_val, max_pos, 1)

    cp = pltpu.make_async_copy(
        mem_ref.at[pl.ds(am, 1), :], scratch_ref, sem)
    cp.start()
    cp.wait()

    use_new = reward_sum > max_val
    mem_out_ref[:] = jnp.where(use_new, states, scratch_ref[:])
    idx_out_ref[:] = jnp.full((1, 1), jnp.where(use_new, reward_sum, max_val),
                              dtype=jnp.float32)


@jax.jit
def kernel(x, maximum_route, memory, index):
    del maximum_route
    xs = x.reshape(T, FEAT)
    idx = index.reshape(MEMORY_LEN)
    idx = jnp.pad(idx, (0, _PAD), constant_values=jnp.inf).reshape(_ROWS, FEAT)
    mem = memory.reshape(MEMORY_LEN, FEAT)

    mem_out, idx_out = pl.pallas_call(
        _queue_kernel,
        in_specs=[
            pl.BlockSpec(memory_space=pltpu.VMEM),
            pl.BlockSpec(memory_space=pltpu.VMEM),
            pl.BlockSpec(memory_space=pl.ANY),
        ],
        out_specs=[
            pl.BlockSpec(memory_space=pltpu.VMEM),
            pl.BlockSpec(memory_space=pltpu.VMEM),
        ],
        out_shape=[
            jax.ShapeDtypeStruct((1, FEAT), jnp.float32),
            jax.ShapeDtypeStruct((1, 1), jnp.float32),
        ],
        scratch_shapes=[
            pltpu.VMEM((1, FEAT), jnp.float32),
            pltpu.SemaphoreType.DMA,
        ],
    )(xs, idx, mem)

    return mem_out.reshape(1, 1, FEAT), idx_out.reshape(1, 1, 1)
